# baseline (speedup 1.0000x reference)
import functools

import jax
import jax.numpy as jnp
from jax import lax
from jax.experimental import pallas as pl
from jax.experimental.pallas import tpu as pltpu
from jax.experimental.pallas import tpu_sc as plsc

_CUTOFF = 5.0


def _silu(x):
    return x / (1.0 + jnp.exp(-x))


def _xlin_body(x_ref, w_ref, o_ref):
    o_ref[...] = jnp.dot(x_ref[...], w_ref[...],
                         preferred_element_type=jnp.float32)


def _tc_xlin(node, w_pre, bm):
    n, d = node.shape
    return pl.pallas_call(
        _xlin_body,
        grid=(n // bm,),
        in_specs=[
            pl.BlockSpec((bm, d), lambda i: (i, 0)),
            pl.BlockSpec((d, d), lambda i: (0, 0)),
        ],
        out_specs=pl.BlockSpec((bm, d), lambda i: (i, 0)),
        out_shape=jax.ShapeDtypeStruct((n, d), jnp.float32),
    )(node, w_pre)


def _wgen_body(bm, rb, r_ref, ee_ref, fr_ref, w1_ref, b1_ref, w2_ref,
               b2_ref, w3_ref, b3_ref, o_ref):
    nl = 128
    r = r_ref[0]
    r3 = r[:, None, :]
    fr = fr_ref[...][None, :, None]
    bes = jnp.sin(r3 * fr) * (jnp.sqrt(2.0 / _CUTOFF) / r3)
    u = r / _CUTOFF
    u2 = u * u
    u6 = u2 * u2 * u2
    env = 1.0 - 28.0 * u6 + 48.0 * u6 * u - 21.0 * u6 * u2
    env = jnp.where(u < 1.0, env, 0.0)
    scale = env * ee_ref[0]
    aug = jnp.concatenate([bes, scale[:, None, :]], axis=1)
    t = jnp.swapaxes(aug, 1, 2).reshape(bm * nl, rb + 1)
    x = t[:, :rb]
    sc = t[:, rb:]
    h = _silu(jnp.dot(x, w1_ref[...],
                      preferred_element_type=jnp.float32) + b1_ref[...])
    h = _silu(jnp.dot(h, w2_ref[...],
                      preferred_element_type=jnp.float32) + b2_ref[...])
    g = h * sc
    o_ref[...] = (jnp.dot(g, w3_ref[...], preferred_element_type=jnp.float32)
                  + jnp.dot(sc, b3_ref[...],
                            preferred_element_type=jnp.float32))


def _tc_wgen(dist, ee, freqs, w1, b1, w2, b2, w3, b3, bm):
    e = dist.shape[0]
    rb = freqs.shape[0]
    d = w3.shape[1]
    nl = 128
    nblk = e // (nl * bm)
    r2 = dist.reshape(nblk, bm, nl)
    ee2 = ee.reshape(nblk, bm, nl)
    full = lambda i: (0, 0)
    return pl.pallas_call(
        functools.partial(_wgen_body, bm, rb),
        grid=(nblk,),
        in_specs=[
            pl.BlockSpec((1, bm, nl), lambda i: (i, 0, 0)),
            pl.BlockSpec((1, bm, nl), lambda i: (i, 0, 0)),
            pl.BlockSpec((rb,), lambda i: (0,)),
            pl.BlockSpec((rb, rb), full),
            pl.BlockSpec((1, rb), full),
            pl.BlockSpec((rb, rb), full),
            pl.BlockSpec((1, rb), full),
            pl.BlockSpec((rb, d), full),
            pl.BlockSpec((1, d), full),
        ],
        out_specs=pl.BlockSpec((bm * nl, d), lambda i: (i, 0)),
        out_shape=jax.ShapeDtypeStruct((e, d), jnp.float32),
    )(r2, ee2, freqs, w1, b1.reshape(1, rb), w2, b2.reshape(1, rb),
      w3, b3.reshape(1, d))


def _sc_message(xlin, wfull, nbr, cent, n_nodes, eoff, chunk):
    e, d = wfull.shape
    ncores, nsub = 2, 16
    nworkers = ncores * nsub
    epw = e // nworkers
    nchunks = epw // chunk
    assert nchunks >= 3 and chunk * nchunks == epw
    assert chunk % 8 == 0 and chunk <= 128
    rows_per_tile = (n_nodes // nsub) // 8 * 8
    tail_rows = n_nodes - rows_per_tile * nsub
    nz_full = rows_per_tile // chunk
    nz_rem = rows_per_tile - nz_full * chunk

    mesh = plsc.VectorSubcoreMesh(core_axis_name="c", subcore_axis_name="s")

    @functools.partial(
        pl.kernel,
        mesh=mesh,
        out_type=jax.ShapeDtypeStruct((ncores, n_nodes, d), jnp.float32),
        scratch_types=[
            pltpu.VMEM((2, chunk), jnp.int32),
            pltpu.VMEM((2, chunk), jnp.int32),
            pltpu.VMEM((2, chunk), jnp.int32),
            pltpu.VMEM((2, chunk, d), jnp.float32),
            pltpu.VMEM((2, chunk, d), jnp.float32),
            pltpu.VMEM_SHARED((n_nodes, d), jnp.float32),
            pltpu.SemaphoreType.DMA,
            pltpu.SemaphoreType.DMA,
            pltpu.SemaphoreType.DMA,
            pltpu.SemaphoreType.DMA,
            pltpu.SemaphoreType.DMA,
            pltpu.SemaphoreType.DMA,
            pltpu.SemaphoreType.DMA,
            pltpu.SemaphoreType.DMA,
        ],
    )
    def k(xlin_hbm, w_hbm, nbr_hbm, cent_hbm, out_hbm,
          nbr_v, cent_v, cent_s, rows_v, w_v, acc,
          sg0, sg1, sw0, sw1, si0, si1, ss0, ss1):
        c = lax.axis_index("c")
        s = lax.axis_index("s")
        wid = c * nsub + s

        wbase = wid * epw
        ebase = eoff + wid * epw
        sgl = (sg0, sg1)
        swl = (sw0, sw1)
        sil = (si0, si1)
        ssl = (ss0, ss1)

        def idx_issue(ci, b):
            off = ebase + ci * chunk
            pltpu.async_copy(nbr_hbm.at[pl.ds(off, chunk)],
                             nbr_v.at[b], sil[b])
            pltpu.async_copy(cent_hbm.at[pl.ds(off, chunk)],
                             cent_v.at[b], sil[b])

        def idx_wait(b):
            pltpu.make_async_copy(nbr_hbm.at[pl.ds(ebase, chunk)],
                                  nbr_v.at[b], sil[b]).wait()
            pltpu.make_async_copy(cent_hbm.at[pl.ds(ebase, chunk)],
                                  cent_v.at[b], sil[b]).wait()

        def fetch_issue(ci, b):
            pltpu.async_copy(xlin_hbm.at[nbr_v.at[b]], rows_v.at[b], sgl[b])
            pltpu.async_copy(w_hbm.at[pl.ds(wbase + ci * chunk, chunk)],
                             w_v.at[b], swl[b])

        def fetch_wait(b):
            pltpu.make_async_copy(xlin_hbm.at[nbr_v.at[b]],
                                  rows_v.at[b], sgl[b]).wait()
            pltpu.make_async_copy(w_hbm.at[pl.ds(wbase, chunk)],
                                  w_v.at[b], swl[b]).wait()

        def cent_save(b):
            starts = list(range(0, chunk - 15, 16))
            if chunk % 16:
                starts.append(chunk - 16)
            for st in starts:
                sl = pl.ds(st, 16)
                cent_s[b, sl] = cent_v[b, sl]

        def multiply(b):
            @plsc.parallel_loop(0, chunk, 2, unroll=4)
            def mrow(r):
                for rr in range(2):
                    for j in range(d // 16):
                        sl = pl.ds(j * 16, 16)
                        w_v[b, r + rr, sl] = (rows_v[b, r + rr, sl]
                                              * w_v[b, r + rr, sl])

        def scatter_issue(b):
            pltpu.async_copy(w_v.at[b], acc.at[cent_s.at[b]], ssl[b],
                             add=True)

        def scatter_wait(b):
            pltpu.make_async_copy(w_v.at[b], acc.at[cent_s.at[b]],
                                  ssl[b]).wait()

        pltpu.sync_copy(nbr_hbm.at[pl.ds(ebase, chunk)], nbr_v.at[0])
        pltpu.sync_copy(cent_hbm.at[pl.ds(ebase, chunk)], cent_v.at[0])
        idx_issue(1, 1)
        fetch_issue(0, 0)

        zero = jnp.zeros((16,), jnp.float32)

        def zrow(i, carry):
            for j in range(d // 16):
                w_v[1, i, pl.ds(j * 16, 16)] = zero
            return carry

        lax.fori_loop(0, chunk, zrow, 0)
        base_row = s * rows_per_tile
        for rblk in range(nz_full):
            pltpu.sync_copy(w_v.at[1],
                            acc.at[pl.ds(base_row + rblk * chunk, chunk)])
        if nz_rem:
            pltpu.sync_copy(w_v.at[1].at[pl.ds(0, nz_rem)],
                            acc.at[pl.ds(base_row + nz_full * chunk, nz_rem)])

        @pl.when(s == nsub - 1)
        def _zero_tail():
            pltpu.sync_copy(w_v.at[1].at[pl.ds(0, tail_rows)],
                            acc.at[pl.ds(rows_per_tile * nsub, tail_rows)])

        plsc.subcore_barrier()

        fetch_wait(0)
        cent_save(0)
        idx_issue(2, 0)
        idx_wait(1)
        fetch_issue(1, 1)
        multiply(0)
        scatter_issue(0)

        fetch_wait(1)
        cent_save(1)
        idx_issue(3, 1)
        scatter_wait(0)
        idx_wait(0)
        fetch_issue(2, 0)
        multiply(1)
        scatter_issue(1)

        full_pairs = (nchunks - 2) // 2

        def pair(kk, carry):
            for b in (0, 1):
                i = 2 * kk + b
                nb = 1 - b
                fetch_wait(b)
                cent_save(b)

                @pl.when(i + 2 < nchunks)
                def _idx():
                    idx_issue(i + 2, b)

                scatter_wait(nb)

                @pl.when(i + 1 < nchunks)
                def _next():
                    idx_wait(nb)
                    fetch_issue(i + 1, nb)

                multiply(b)
                scatter_issue(b)
            return carry

        lax.fori_loop(1, 1 + full_pairs, pair, 0)

        if nchunks % 2:
            fetch_wait(0)
            scatter_wait(1)
            multiply(0)
            pltpu.async_copy(w_v.at[0], acc.at[cent_v.at[0]], ss0, add=True)
            pltpu.make_async_copy(w_v.at[0], acc.at[cent_v.at[0]], ss0).wait()
        else:
            scatter_wait(1)
        plsc.subcore_barrier()
        pltpu.sync_copy(acc.at[pl.ds(base_row, rows_per_tile)],
                        out_hbm.at[c, pl.ds(base_row, rows_per_tile)])

        @pl.when(s == nsub - 1)
        def _write_tail():
            pltpu.sync_copy(acc.at[pl.ds(rows_per_tile * nsub, tail_rows)],
                            out_hbm.at[c, pl.ds(rows_per_tile * nsub,
                                                tail_rows)])

    return k(xlin, wfull, nbr, cent)


def _final_body(nparts, *refs):
    parts = refs[:nparts]
    nb_ref, wpost_ref, wself_ref, o_ref = refs[nparts:]
    t = parts[0][0] + parts[0][1]
    for p in parts[1:]:
        t = t + p[0] + p[1]
    y = (jnp.dot(t, wpost_ref[...], preferred_element_type=jnp.float32)
         + jnp.dot(nb_ref[...], wself_ref[...],
                   preferred_element_type=jnp.float32))
    o_ref[...] = _silu(y)


def _tc_final(parts, node, w_post, w_self, bm):
    n, d = node.shape
    full = lambda i: (0, 0)
    blk = pl.BlockSpec((bm, d), lambda i: (i, 0))
    pblk = pl.BlockSpec((2, bm, d), lambda i: (0, i, 0))
    return pl.pallas_call(
        functools.partial(_final_body, len(parts)),
        grid=(n // bm,),
        in_specs=[pblk] * len(parts) + [
            blk,
            pl.BlockSpec((d, d), full),
            pl.BlockSpec((d, d), full)],
        out_specs=blk,
        out_shape=jax.ShapeDtypeStruct((n, d), jnp.float32),
    )(*parts, node, w_post, w_self)


def kernel(node_embeddings, Z_embeddings, neighbour_distances, edge_embedding,
           graph, bessel_freqs, mlp_w1, mlp_b1, mlp_w2, mlp_b2, mlp_w3,
           mlp_b3, w_pre, w_post, w_self):
    n, d = node_embeddings.shape
    e = graph.shape[1]
    cent = graph[0]
    nbr = graph[1]
    x_lin = _tc_xlin(node_embeddings, w_pre, bm=400)
    cu = e // (32 * 80)
    u3 = min(cu - cu // 2, 56)
    u1 = (cu - u3) * 2 // 5
    u2 = cu - u3 - u1
    splits = [(u1, 80), (u2, 80), (u3, 80)]
    parts = []
    off = 0
    for nu, chunk in splits:
        sz = nu * 32 * 80
        e128 = sz // 128
        bm = next(bb for bb in range(64, 0, -1) if e128 % bb == 0)
        sl = slice(off, off + sz)
        w_h = _tc_wgen(neighbour_distances[sl], edge_embedding[sl],
                       bessel_freqs, mlp_w1, mlp_b1, mlp_w2, mlp_b2,
                       mlp_w3, mlp_b3, bm=bm)
        parts.append(_sc_message(x_lin, w_h, nbr, cent, n,
                                 eoff=off, chunk=chunk))
        off += sz
    return _tc_final(parts, node_embeddings, w_post, w_self, bm=400)


# speedup vs baseline: 1.0513x; 1.0513x over previous
import functools

import jax
import jax.numpy as jnp
from jax import lax
from jax.experimental import pallas as pl
from jax.experimental.pallas import tpu as pltpu
from jax.experimental.pallas import tpu_sc as plsc

_CUTOFF = 5.0


def _silu(x):
    return x / (1.0 + jnp.exp(-x))


def _xlin_body(x_ref, w_ref, o_ref):
    o_ref[...] = jnp.dot(x_ref[...], w_ref[...],
                         preferred_element_type=jnp.float32)


def _tc_xlin(node, w_pre, bm):
    n, d = node.shape
    return pl.pallas_call(
        _xlin_body,
        grid=(n // bm,),
        in_specs=[
            pl.BlockSpec((bm, d), lambda i: (i, 0)),
            pl.BlockSpec((d, d), lambda i: (0, 0)),
        ],
        out_specs=pl.BlockSpec((bm, d), lambda i: (i, 0)),
        out_shape=jax.ShapeDtypeStruct((n, d), jnp.float32),
    )(node, w_pre)


def _wgen_body(bm, rb, r_ref, ee_ref, fr_ref, w1_ref, b1_ref, w2_ref,
               b2_ref, w3_ref, b3_ref, o_ref):
    nl = 128
    r = r_ref[0]
    r3 = r[:, None, :]
    fr = fr_ref[...][None, :, None]
    bes = jnp.sin(r3 * fr) * (jnp.sqrt(2.0 / _CUTOFF) / r3)
    u = r / _CUTOFF
    u2 = u * u
    u6 = u2 * u2 * u2
    env = 1.0 - 28.0 * u6 + 48.0 * u6 * u - 21.0 * u6 * u2
    env = jnp.where(u < 1.0, env, 0.0)
    scale = env * ee_ref[0]
    aug = jnp.concatenate([bes, scale[:, None, :]], axis=1)
    t = jnp.swapaxes(aug, 1, 2).reshape(bm * nl, rb + 1)
    x = t[:, :rb]
    sc = t[:, rb:]
    h = _silu(jnp.dot(x, w1_ref[...],
                      preferred_element_type=jnp.float32) + b1_ref[...])
    h = _silu(jnp.dot(h, w2_ref[...],
                      preferred_element_type=jnp.float32) + b2_ref[...])
    g = h * sc
    o_ref[...] = (jnp.dot(g, w3_ref[...], preferred_element_type=jnp.float32)
                  + jnp.dot(sc, b3_ref[...],
                            preferred_element_type=jnp.float32))


def _tc_wgen(dist, ee, freqs, w1, b1, w2, b2, w3, b3, bm):
    e = dist.shape[0]
    rb = freqs.shape[0]
    d = w3.shape[1]
    nl = 128
    nblk = e // (nl * bm)
    r2 = dist.reshape(nblk, bm, nl)
    ee2 = ee.reshape(nblk, bm, nl)
    full = lambda i: (0, 0)
    return pl.pallas_call(
        functools.partial(_wgen_body, bm, rb),
        grid=(nblk,),
        in_specs=[
            pl.BlockSpec((1, bm, nl), lambda i: (i, 0, 0)),
            pl.BlockSpec((1, bm, nl), lambda i: (i, 0, 0)),
            pl.BlockSpec((rb,), lambda i: (0,)),
            pl.BlockSpec((rb, rb), full),
            pl.BlockSpec((1, rb), full),
            pl.BlockSpec((rb, rb), full),
            pl.BlockSpec((1, rb), full),
            pl.BlockSpec((rb, d), full),
            pl.BlockSpec((1, d), full),
        ],
        out_specs=pl.BlockSpec((bm * nl, d), lambda i: (i, 0)),
        out_shape=jax.ShapeDtypeStruct((e, d), jnp.float32),
    )(r2, ee2, freqs, w1, b1.reshape(1, rb), w2, b2.reshape(1, rb),
      w3, b3.reshape(1, d))


def _sc_message(xlin, wfull, nbr, cent, n_nodes, eoff, chunk):
    e, d = wfull.shape
    ncores, nsub = 2, 16
    nworkers = ncores * nsub
    epw = e // nworkers
    nchunks = epw // chunk
    assert nchunks >= 3 and chunk * nchunks == epw
    assert chunk % 8 == 0 and chunk <= 128
    rows_per_tile = (n_nodes // nsub) // 8 * 8
    tail_rows = n_nodes - rows_per_tile * nsub
    nz_full = rows_per_tile // chunk
    nz_rem = rows_per_tile - nz_full * chunk

    mesh = plsc.VectorSubcoreMesh(core_axis_name="c", subcore_axis_name="s")

    @functools.partial(
        pl.kernel,
        mesh=mesh,
        out_type=jax.ShapeDtypeStruct((ncores, n_nodes, d), jnp.float32),
        scratch_types=[
            pltpu.VMEM((2, chunk), jnp.int32),
            pltpu.VMEM((2, chunk), jnp.int32),
            pltpu.VMEM((2, chunk), jnp.int32),
            pltpu.VMEM((2, chunk, d), jnp.float32),
            pltpu.VMEM((2, chunk, d), jnp.float32),
            pltpu.VMEM_SHARED((n_nodes, d), jnp.float32),
            pltpu.SemaphoreType.DMA,
            pltpu.SemaphoreType.DMA,
            pltpu.SemaphoreType.DMA,
            pltpu.SemaphoreType.DMA,
            pltpu.SemaphoreType.DMA,
            pltpu.SemaphoreType.DMA,
            pltpu.SemaphoreType.DMA,
            pltpu.SemaphoreType.DMA,
        ],
    )
    def k(xlin_hbm, w_hbm, nbr_hbm, cent_hbm, out_hbm,
          nbr_v, cent_v, cent_s, rows_v, w_v, acc,
          sg0, sg1, sw0, sw1, si0, si1, ss0, ss1):
        c = lax.axis_index("c")
        s = lax.axis_index("s")
        wid = c * nsub + s

        wbase = wid * epw
        ebase = eoff + wid * epw
        sgl = (sg0, sg1)
        swl = (sw0, sw1)
        sil = (si0, si1)
        ssl = (ss0, ss1)

        def idx_issue(ci, b):
            off = ebase + ci * chunk
            pltpu.async_copy(nbr_hbm.at[pl.ds(off, chunk)],
                             nbr_v.at[b], sil[b])
            pltpu.async_copy(cent_hbm.at[pl.ds(off, chunk)],
                             cent_v.at[b], sil[b])

        def idx_wait(b):
            pltpu.make_async_copy(nbr_hbm.at[pl.ds(ebase, chunk)],
                                  nbr_v.at[b], sil[b]).wait()
            pltpu.make_async_copy(cent_hbm.at[pl.ds(ebase, chunk)],
                                  cent_v.at[b], sil[b]).wait()

        def fetch_issue(ci, b):
            pltpu.async_copy(xlin_hbm.at[nbr_v.at[b]], rows_v.at[b], sgl[b])
            pltpu.async_copy(w_hbm.at[pl.ds(wbase + ci * chunk, chunk)],
                             w_v.at[b], swl[b])

        def fetch_wait(b):
            pltpu.make_async_copy(xlin_hbm.at[nbr_v.at[b]],
                                  rows_v.at[b], sgl[b]).wait()
            pltpu.make_async_copy(w_hbm.at[pl.ds(wbase, chunk)],
                                  w_v.at[b], swl[b]).wait()

        def cent_save(b):
            starts = list(range(0, chunk - 15, 16))
            if chunk % 16:
                starts.append(chunk - 16)
            for st in starts:
                sl = pl.ds(st, 16)
                cent_s[b, sl] = cent_v[b, sl]

        def multiply(b):
            @plsc.parallel_loop(0, chunk, 2, unroll=4)
            def mrow(r):
                for rr in range(2):
                    for j in range(d // 16):
                        sl = pl.ds(j * 16, 16)
                        w_v[b, r + rr, sl] = (rows_v[b, r + rr, sl]
                                              * w_v[b, r + rr, sl])

        def scatter_issue(b):
            pltpu.async_copy(w_v.at[b], acc.at[cent_s.at[b]], ssl[b],
                             add=True)

        def scatter_wait(b):
            pltpu.make_async_copy(w_v.at[b], acc.at[cent_s.at[b]],
                                  ssl[b]).wait()

        pltpu.sync_copy(nbr_hbm.at[pl.ds(ebase, chunk)], nbr_v.at[0])
        pltpu.sync_copy(cent_hbm.at[pl.ds(ebase, chunk)], cent_v.at[0])
        idx_issue(1, 1)
        fetch_issue(0, 0)

        zero = jnp.zeros((16,), jnp.float32)

        def zrow(i, carry):
            for j in range(d // 16):
                w_v[1, i, pl.ds(j * 16, 16)] = zero
            return carry

        lax.fori_loop(0, chunk, zrow, 0)
        base_row = s * rows_per_tile
        for rblk in range(nz_full):
            pltpu.sync_copy(w_v.at[1],
                            acc.at[pl.ds(base_row + rblk * chunk, chunk)])
        if nz_rem:
            pltpu.sync_copy(w_v.at[1].at[pl.ds(0, nz_rem)],
                            acc.at[pl.ds(base_row + nz_full * chunk, nz_rem)])

        @pl.when(s == nsub - 1)
        def _zero_tail():
            pltpu.sync_copy(w_v.at[1].at[pl.ds(0, tail_rows)],
                            acc.at[pl.ds(rows_per_tile * nsub, tail_rows)])

        plsc.subcore_barrier()

        fetch_wait(0)
        cent_save(0)
        idx_issue(2, 0)
        idx_wait(1)
        fetch_issue(1, 1)
        multiply(0)
        scatter_issue(0)

        fetch_wait(1)
        cent_save(1)
        idx_issue(3, 1)
        scatter_wait(0)
        idx_wait(0)
        fetch_issue(2, 0)
        multiply(1)
        scatter_issue(1)

        full_pairs = (nchunks - 2) // 2

        def pair(kk, carry):
            for b in (0, 1):
                i = 2 * kk + b
                nb = 1 - b
                fetch_wait(b)
                cent_save(b)

                @pl.when(i + 2 < nchunks)
                def _idx():
                    idx_issue(i + 2, b)

                scatter_wait(nb)

                @pl.when(i + 1 < nchunks)
                def _next():
                    idx_wait(nb)
                    fetch_issue(i + 1, nb)

                multiply(b)
                scatter_issue(b)
            return carry

        lax.fori_loop(1, 1 + full_pairs, pair, 0)

        if nchunks % 2:
            fetch_wait(0)
            scatter_wait(1)
            multiply(0)
            pltpu.async_copy(w_v.at[0], acc.at[cent_v.at[0]], ss0, add=True)
            pltpu.make_async_copy(w_v.at[0], acc.at[cent_v.at[0]], ss0).wait()
        else:
            scatter_wait(1)
        plsc.subcore_barrier()
        pltpu.sync_copy(acc.at[pl.ds(base_row, rows_per_tile)],
                        out_hbm.at[c, pl.ds(base_row, rows_per_tile)])

        @pl.when(s == nsub - 1)
        def _write_tail():
            pltpu.sync_copy(acc.at[pl.ds(rows_per_tile * nsub, tail_rows)],
                            out_hbm.at[c, pl.ds(rows_per_tile * nsub,
                                                tail_rows)])

    return k(xlin, wfull, nbr, cent)


def _final_body(nparts, *refs):
    parts = refs[:nparts]
    nb_ref, wpost_ref, wself_ref, o_ref = refs[nparts:]
    t = parts[0][0] + parts[0][1]
    for p in parts[1:]:
        t = t + p[0] + p[1]
    y = (jnp.dot(t, wpost_ref[...], preferred_element_type=jnp.float32)
         + jnp.dot(nb_ref[...], wself_ref[...],
                   preferred_element_type=jnp.float32))
    o_ref[...] = _silu(y)


def _tc_final(parts, node, w_post, w_self, bm):
    n, d = node.shape
    full = lambda i: (0, 0)
    blk = pl.BlockSpec((bm, d), lambda i: (i, 0))
    pblk = pl.BlockSpec((2, bm, d), lambda i: (0, i, 0))
    return pl.pallas_call(
        functools.partial(_final_body, len(parts)),
        grid=(n // bm,),
        in_specs=[pblk] * len(parts) + [
            blk,
            pl.BlockSpec((d, d), full),
            pl.BlockSpec((d, d), full)],
        out_specs=blk,
        out_shape=jax.ShapeDtypeStruct((n, d), jnp.float32),
    )(*parts, node, w_post, w_self)


def kernel(node_embeddings, Z_embeddings, neighbour_distances, edge_embedding,
           graph, bessel_freqs, mlp_w1, mlp_b1, mlp_w2, mlp_b2, mlp_w3,
           mlp_b3, w_pre, w_post, w_self):
    n, d = node_embeddings.shape
    e = graph.shape[1]
    cent = graph[0]
    nbr = graph[1]
    x_lin = _tc_xlin(node_embeddings, w_pre, bm=400)
    cu = e // (32 * 80)
    u1 = (cu + 2) // 3
    u2 = u1
    u3 = cu - u1 - u2
    splits = [(u1, 80), (u2, 80), (u3, 80)]
    parts = []
    off = 0
    for nu, chunk in splits:
        sz = nu * 32 * 80
        e128 = sz // 128
        bm = next(bb for bb in range(64, 0, -1) if e128 % bb == 0)
        sl = slice(off, off + sz)
        w_h = _tc_wgen(neighbour_distances[sl], edge_embedding[sl],
                       bessel_freqs, mlp_w1, mlp_b1, mlp_w2, mlp_b2,
                       mlp_w3, mlp_b3, bm=bm)
        parts.append(_sc_message(x_lin, w_h, nbr, cent, n,
                                 eoff=off, chunk=chunk))
        off += sz
    return _tc_final(parts, node_embeddings, w_post, w_self, bm=400)
